# Initial kernel scaffold; baseline (speedup 1.0000x reference)
import jax
import jax.numpy as jnp
from jax.experimental import pallas as pl


def kernel(positions, numbers, edge_index, edge_shift, batch, cell, cutoff, covalent_radii, a, b, a0, p):
    raise NotImplementedError("write your pallas kernel here")


# speedup vs baseline: 93.7790x; 93.7790x over previous
import functools

import jax
import jax.numpy as jnp
import numpy as np
from jax import lax
from jax.experimental import pallas as pl
from jax.experimental.pallas import tpu as pltpu
from jax.experimental.pallas import tpu_sc as plsc

KE = 14.399645
NC = 2
NS = 16
NW = NC * NS
L = 16
C = 512
JB = 128
NJ = C // JB


def _sc_body(nchunks, Bpad, NPA, refs):
    (nx_h, ny_h, nz_h, nzf_h, nzq_h, nrc_h, nbt_h,
     src_h, dst_h, sx_h, sy_h, sz_h, par_h, zer_h, zbr_h,
     fout, ebout, sbout,
     accx, accy, accz, ebsh, ssh0, ssh1, ssh2, ssh3, ssh4, ssh5, ssh6,
     ssh7, ssh8,
     sidx, didx, bsix, bdix, shxv, shyv, shzv,
     sxv, syv, szv, szfv, szqv, srcvv,
     dxv, dyv, dzv, dzfv, dzqv, drcvv,
     gsx, gsy, gsz, gdx, gdy, gdz,
     engst, st0, st1, st2, st3, st4, st5, st6, st7, st8,
     parv, sem) = refs
    cid = lax.axis_index("c")
    sid = lax.axis_index("s")
    wid = sid * NC + cid

    pltpu.sync_copy(par_h, parv)

    def splat(k):
        return parv[pl.ds(k * L, L)]

    A0, A1, A2, A3 = splat(0), splat(1), splat(2), splat(3)
    B0, B1, B2, B3 = splat(4), splat(5), splat(6), splat(7)
    a0v, cutv = splat(8), splat(9)
    AB0, AB1, AB2, AB3 = A0 * B0, A1 * B1, A2 * B2, A3 * B3
    inva0 = 1.0 / a0v

    @pl.when(sid == 0)
    def _():
        pltpu.sync_copy(zer_h, accx)
        pltpu.sync_copy(zer_h, accy)
        pltpu.sync_copy(zer_h, accz)
        pltpu.sync_copy(zbr_h, ebsh)
        for ssh in (ssh0, ssh1, ssh2, ssh3, ssh4, ssh5, ssh6, ssh7, ssh8):
            pltpu.sync_copy(zbr_h, ssh)

    plsc.subcore_barrier()

    @pl.loop(0, nchunks)
    def _(g):
        eb = (wid * nchunks + g) * C
        rb = (wid * nchunks + g) * NJ
        pltpu.sync_copy(src_h.at[pl.ds(rb, NJ)], sidx)
        pltpu.sync_copy(dst_h.at[pl.ds(rb, NJ)], didx)
        pltpu.sync_copy(sx_h.at[pl.ds(eb, C)], shxv)
        pltpu.sync_copy(sy_h.at[pl.ds(eb, C)], shyv)
        pltpu.sync_copy(sz_h.at[pl.ds(eb, C)], shzv)

        @pl.loop(0, NJ)
        def _(j):
            osl = pl.ds(j * JB, JB)
            isl = sidx.at[j]
            idl = didx.at[j]
            descs = [
                pltpu.async_copy(nx_h.at[isl], sxv.at[osl], sem),
                pltpu.async_copy(ny_h.at[isl], syv.at[osl], sem),
                pltpu.async_copy(nz_h.at[isl], szv.at[osl], sem),
                pltpu.async_copy(nzf_h.at[isl], szfv.at[osl], sem),
                pltpu.async_copy(nzq_h.at[isl], szqv.at[osl], sem),
                pltpu.async_copy(nrc_h.at[isl], srcvv.at[osl], sem),
                pltpu.async_copy(nbt_h.at[isl], bsix.at[j], sem),
                pltpu.async_copy(nx_h.at[idl], dxv.at[osl], sem),
                pltpu.async_copy(ny_h.at[idl], dyv.at[osl], sem),
                pltpu.async_copy(nz_h.at[idl], dzv.at[osl], sem),
                pltpu.async_copy(nzf_h.at[idl], dzfv.at[osl], sem),
                pltpu.async_copy(nzq_h.at[idl], dzqv.at[osl], sem),
                pltpu.async_copy(nrc_h.at[idl], drcvv.at[osl], sem),
                pltpu.async_copy(nbt_h.at[idl], bdix.at[j], sem),
            ]
            for dsc in descs:
                dsc.wait()

        @pl.loop(0, C // L)
        def _(v):
            sl16 = pl.ds(v * L, L)
            sxp = sxv[sl16]
            syp = syv[sl16]
            szp = szv[sl16]
            szf = szfv[sl16]
            szq = szqv[sl16]
            srcv = srcvv[sl16]
            dxp = dxv[sl16]
            dyp = dyv[sl16]
            dzp = dzv[sl16]
            dzf = dzfv[sl16]
            dzq = dzqv[sl16]
            drcv = drcvv[sl16]
            shx = shxv[sl16]
            shy = shyv[sl16]
            shz = shzv[sl16]

            vx = dxp - sxp + shx
            vy = dyp - syp + shy
            vz = dzp - szp + shz
            r2 = vx * vx + vy * vy + vz * vz
            ib = lax.bitcast_convert_type(r2, jnp.int32)
            y = lax.bitcast_convert_type(
                jnp.full((L,), 0x5F3759DF, jnp.int32) - (ib >> 1), jnp.float32)
            y = y * (1.5 - 0.5 * r2 * y * y)
            y = y * (1.5 - 0.5 * r2 * y * y)
            y = y * (1.5 - 0.5 * r2 * y * y)
            inv_r = y
            r = r2 * inv_r

            Z = szf * dzf
            sumzp = szq + dzq
            x = r * sumzp * inva0
            rbond = srcv + drcv
            rc = jnp.minimum(cutv, rbond)
            invrc = 1.0 / rc
            u = jnp.minimum(r * invrc, 1.0)
            u2 = u * u
            u4 = u2 * u2
            u5 = u4 * u
            u6 = u5 * u
            u7 = u6 * u
            env = 1.0 - 28.0 * u6 + 48.0 * u7 - 21.0 * u6 * u2
            denv = (-168.0 * u5 + 336.0 * u6 - 168.0 * u7) * invrc

            e0 = jnp.exp(B0 * x)
            e1 = jnp.exp(B1 * x)
            e2 = jnp.exp(B2 * x)
            e3 = jnp.exp(B3 * x)
            phi = A0 * e0 + A1 * e1 + A2 * e2 + A3 * e3
            dphi = AB0 * e0 + AB1 * e1 + AB2 * e2 + AB3 * e3

            kez = KE * Z
            eng = kez * inv_r * phi * env
            term = dphi * env * sumzp * inva0 + phi * denv - phi * env * inv_r
            s = 0.5 * kez * inv_r * inv_r * term
            gx = vx * s
            gy = vy * s
            gz = vz * s

            gsx[sl16] = gx
            gsy[sl16] = gy
            gsz[sl16] = gz
            gdx[sl16] = -gx
            gdy[sl16] = -gy
            gdz[sl16] = -gz

            engst[sl16] = eng
            st0[sl16] = vx * gx
            st1[sl16] = vx * gy
            st2[sl16] = vx * gz
            st3[sl16] = vy * gx
            st4[sl16] = vy * gy
            st5[sl16] = vy * gz
            st6[sl16] = vz * gx
            st7[sl16] = vz * gy
            st8[sl16] = vz * gz

        @pl.loop(0, NJ)
        def _(j):
            sl = pl.ds(j * JB, JB)
            isl = sidx.at[j]
            idl = didx.at[j]
            pltpu.sync_copy(gsx.at[sl], accx.at[isl], add=True)
            pltpu.sync_copy(gsy.at[sl], accy.at[isl], add=True)
            pltpu.sync_copy(gsz.at[sl], accz.at[isl], add=True)
            pltpu.sync_copy(gdx.at[sl], accx.at[idl], add=True)
            pltpu.sync_copy(gdy.at[sl], accy.at[idl], add=True)
            pltpu.sync_copy(gdz.at[sl], accz.at[idl], add=True)
            pltpu.sync_copy(engst.at[sl], ebsh.at[bdix.at[j]], add=True)
            bsl = bsix.at[j]
            pltpu.sync_copy(st0.at[sl], ssh0.at[bsl], add=True)
            pltpu.sync_copy(st1.at[sl], ssh1.at[bsl], add=True)
            pltpu.sync_copy(st2.at[sl], ssh2.at[bsl], add=True)
            pltpu.sync_copy(st3.at[sl], ssh3.at[bsl], add=True)
            pltpu.sync_copy(st4.at[sl], ssh4.at[bsl], add=True)
            pltpu.sync_copy(st5.at[sl], ssh5.at[bsl], add=True)
            pltpu.sync_copy(st6.at[sl], ssh6.at[bsl], add=True)
            pltpu.sync_copy(st7.at[sl], ssh7.at[bsl], add=True)
            pltpu.sync_copy(st8.at[sl], ssh8.at[bsl], add=True)

    plsc.subcore_barrier()

    @pl.when(sid == 0)
    def _():
        pltpu.sync_copy(accx, fout.at[cid * 3 + 0])
        pltpu.sync_copy(accy, fout.at[cid * 3 + 1])
        pltpu.sync_copy(accz, fout.at[cid * 3 + 2])
        pltpu.sync_copy(ebsh, ebout.at[cid])
        pltpu.sync_copy(ssh0, sbout.at[cid * 9 + 0])
        pltpu.sync_copy(ssh1, sbout.at[cid * 9 + 1])
        pltpu.sync_copy(ssh2, sbout.at[cid * 9 + 2])
        pltpu.sync_copy(ssh3, sbout.at[cid * 9 + 3])
        pltpu.sync_copy(ssh4, sbout.at[cid * 9 + 4])
        pltpu.sync_copy(ssh5, sbout.at[cid * 9 + 5])
        pltpu.sync_copy(ssh6, sbout.at[cid * 9 + 6])
        pltpu.sync_copy(ssh7, sbout.at[cid * 9 + 7])
        pltpu.sync_copy(ssh8, sbout.at[cid * 9 + 8])


def _body(nchunks, Bpad, NPA, *refs):
    return _sc_body(nchunks, Bpad, NPA, refs)


def kernel(positions, numbers, edge_index, edge_shift, batch, cell, cutoff,
           covalent_radii, a, b, a0, p):
    N = positions.shape[0]
    E = edge_index.shape[1]
    B = cell.shape[0]
    NP = N + 8
    NPA = ((NP + 7) // 8) * 8
    Bpad = ((B + 15) // 16) * 16
    per_w = C * ((E + NW * C - 1) // (NW * C))
    Ep = NW * per_w
    nchunks = per_w // C

    f32 = jnp.float32
    npad = NPA - N
    zf = numbers.astype(f32)
    padpos = np.zeros((npad, 3), np.float32)
    padpos[1, 0] = 100.0
    nx = jnp.concatenate([positions[:, 0], jnp.asarray(padpos[:, 0])])
    ny = jnp.concatenate([positions[:, 1], jnp.asarray(padpos[:, 1])])
    nz = jnp.concatenate([positions[:, 2], jnp.asarray(padpos[:, 2])])
    nzf = jnp.concatenate([zf, jnp.ones((npad,), f32)])
    nzq = jnp.concatenate([jnp.power(zf, p), jnp.ones((npad,), f32)])
    nrc = jnp.concatenate([covalent_radii[numbers],
                           jnp.full((npad,), 0.3, f32)])
    nbt = jnp.concatenate([batch, jnp.zeros((npad,), jnp.int32)])

    src = jnp.concatenate([edge_index[0], jnp.full((Ep - E,), N, jnp.int32)])
    dst = jnp.concatenate([edge_index[1], jnp.full((Ep - E,), N + 1, jnp.int32)])
    shp = jnp.concatenate([edge_shift, jnp.zeros((Ep - E, 3), f32)], axis=0)
    src2 = src.reshape(Ep // JB, JB)
    dst2 = dst.reshape(Ep // JB, JB)
    sx = shp[:, 0]
    sy = shp[:, 1]
    sz = shp[:, 2]
    params = jnp.repeat(
        jnp.concatenate([a, b, jnp.reshape(a0, (1,)),
                         jnp.reshape(cutoff, (1,))]), L)
    zer = jnp.zeros((NPA,), f32)
    zbr = jnp.zeros((Bpad,), f32)

    mesh = plsc.VectorSubcoreMesh(core_axis_name="c", subcore_axis_name="s",
                                  num_cores=NC, num_subcores=NS)
    cvec = [pltpu.VMEM((C,), f32) for _ in range(27)]
    run = pl.kernel(
        functools.partial(_body, nchunks, Bpad, NPA),
        out_type=[
            jax.ShapeDtypeStruct((NC * 3, NPA), f32),
            jax.ShapeDtypeStruct((NC, Bpad), f32),
            jax.ShapeDtypeStruct((NC * 9, Bpad), f32),
        ],
        mesh=mesh,
        scratch_types=(
            [pltpu.VMEM_SHARED((NPA,), f32)] * 3
            + [pltpu.VMEM_SHARED((Bpad,), f32)] * 10
            + [pltpu.VMEM((NJ, JB), jnp.int32)] * 4
            + cvec[:3]
            + cvec[3:9]
            + cvec[9:15]
            + cvec[15:21]
            + cvec[21:22]
            + cvec[22:27] + [pltpu.VMEM((C,), f32)] * 4
            + [pltpu.VMEM((10 * L,), f32), pltpu.SemaphoreType.DMA]
        ),
    )
    f_part, e_part, s_part = run(nx, ny, nz, nzf, nzq, nrc, nbt,
                                 src2, dst2, sx, sy, sz, params, zer, zbr)

    forces = (f_part[0:3] + f_part[3:6])[:, :N].T
    energies = 0.5 * (e_part[0] + e_part[1])[:B]
    sraw = (s_part[0:9] + s_part[9:18])[:, :B]
    volume = jnp.linalg.det(cell)
    stress = 0.5 * sraw.T.reshape(B, 3, 3) / volume[:, None, None]
    return (energies, forces, stress)
